# Initial kernel scaffold; baseline (speedup 1.0000x reference)
import jax
import jax.numpy as jnp
from jax.experimental import pallas as pl


def kernel(x, pos_table):
    raise NotImplementedError("write your pallas kernel here")


# speedup vs baseline: 1.9463x; 1.9463x over previous
import jax
import jax.numpy as jnp
from jax.experimental import pallas as pl

_BLOCK_SEQ = 512


def _add_kernel(x_ref, t_ref, o_ref):
    o_ref[...] = x_ref[...] + t_ref[...][None, :, :]


def kernel(x, pos_table):
    batch, seq, dim = x.shape
    bs = min(_BLOCK_SEQ, seq)
    grid = (seq // bs,)
    return pl.pallas_call(
        _add_kernel,
        grid=grid,
        in_specs=[
            pl.BlockSpec((batch, bs, dim), lambda i: (0, i, 0)),
            pl.BlockSpec((bs, dim), lambda i: (i, 0)),
        ],
        out_specs=pl.BlockSpec((batch, bs, dim), lambda i: (0, i, 0)),
        out_shape=jax.ShapeDtypeStruct((batch, seq, dim), x.dtype),
    )(x, pos_table)
